# Initial kernel scaffold; baseline (speedup 1.0000x reference)
import jax
import jax.numpy as jnp
from jax.experimental import pallas as pl


def kernel(x, kan_weight):
    raise NotImplementedError("write your pallas kernel here")


# speedup vs baseline: 334.6671x; 334.6671x over previous
import jax
import jax.numpy as jnp
from jax.experimental import pallas as pl

_IN_F = 256
_OUT_F = 64
_P = 32
_WIDTH = 4.0


def _body(x_ref, w_ref, o_ref):
    p_max = _P - 1
    xs = (x_ref[...] + _WIDTH / 2.0) * (p_max / _WIDTH)
    xsc = jnp.clip(xs, 0.0, float(p_max))
    e_lo = jnp.minimum(xs, 0.0)
    e_hi = jnp.maximum(xs - float(p_max), 0.0)

    dot = lambda a, b: jax.lax.dot(a, b, preferred_element_type=jnp.float32)
    acc = dot(e_lo, w_ref[1] - w_ref[0])
    acc += dot(e_hi, w_ref[p_max] - w_ref[p_max - 1])
    for p in range(_P):
        w_p = jnp.maximum(1.0 - jnp.abs(xsc - float(p)), 0.0)
        acc += dot(w_p, w_ref[p])
    o_ref[...] = acc


def kernel(x, kan_weight):
    b = x.shape[0]
    bt = 512
    wt = jnp.transpose(kan_weight, (1, 0, 2))
    return pl.pallas_call(
        _body,
        grid=(b // bt,),
        in_specs=[
            pl.BlockSpec((bt, _IN_F), lambda i: (i, 0)),
            pl.BlockSpec((_P, _IN_F, _OUT_F), lambda i: (0, 0, 0)),
        ],
        out_specs=pl.BlockSpec((bt, _OUT_F), lambda i: (i, 0)),
        out_shape=jax.ShapeDtypeStruct((b, _OUT_F), jnp.float32),
    )(x, wt)


# speedup vs baseline: 341.3958x; 1.0201x over previous
import jax
import jax.numpy as jnp
from jax.experimental import pallas as pl

_IN_F = 256
_OUT_F = 64
_P = 32
_WIDTH = 4.0


def _body(x_ref, w_ref, o_ref):
    p_max = _P - 1
    xs = (x_ref[...] + _WIDTH / 2.0) * (p_max / _WIDTH)
    xsc = jnp.clip(xs, 0.0, float(p_max))
    e_lo = jnp.minimum(xs, 0.0).astype(jnp.bfloat16)
    e_hi = jnp.maximum(xs - float(p_max), 0.0).astype(jnp.bfloat16)

    dot = lambda a, b: jax.lax.dot(a, b, preferred_element_type=jnp.float32)
    acc = dot(e_lo, w_ref[_P])
    acc += dot(e_hi, w_ref[_P + 1])
    for p in range(_P):
        w_p = jnp.maximum(1.0 - jnp.abs(xsc - float(p)), 0.0)
        acc += dot(w_p.astype(jnp.bfloat16), w_ref[p])
    o_ref[...] = acc


def kernel(x, kan_weight):
    b = x.shape[0]
    bt = 1024
    wt = jnp.transpose(kan_weight, (1, 0, 2))
    wt = jnp.concatenate(
        [wt, (wt[1] - wt[0])[None], (wt[_P - 1] - wt[_P - 2])[None]], axis=0
    ).astype(jnp.bfloat16)
    return pl.pallas_call(
        _body,
        grid=(b // bt,),
        in_specs=[
            pl.BlockSpec((bt, _IN_F), lambda i: (i, 0)),
            pl.BlockSpec((_P + 2, _IN_F, _OUT_F), lambda i: (0, 0, 0)),
        ],
        out_specs=pl.BlockSpec((bt, _OUT_F), lambda i: (i, 0)),
        out_shape=jax.ShapeDtypeStruct((b, _OUT_F), jnp.float32),
    )(x, wt)


# speedup vs baseline: 387.1923x; 1.1341x over previous
import jax
import jax.numpy as jnp
from jax.experimental import pallas as pl

_IN_F = 256
_OUT_F = 64
_P = 32
_WIDTH = 4.0


def _body(x_ref, v_ref, b_ref, o_ref):
    p_max = _P - 1
    xs = (x_ref[...] + _WIDTH / 2.0) * (p_max / _WIDTH)

    dot = lambda a, b: jax.lax.dot(a, b, preferred_element_type=jnp.float32)
    acc = b_ref[0:1, :] + dot(xs, v_ref[0])
    for k in range(1, p_max):
        acc += dot(jnp.maximum(xs - float(k), 0.0), v_ref[k])
    o_ref[...] = acc


def kernel(x, kan_weight):
    b = x.shape[0]
    bt = 1024
    wt = jnp.transpose(kan_weight, (1, 0, 2))
    v = jnp.concatenate(
        [
            (wt[1] - wt[0])[None],
            wt[2:] - 2.0 * wt[1:-1] + wt[:-2],
        ],
        axis=0,
    )
    bias = jnp.broadcast_to(jnp.sum(wt[0], axis=0)[None, :], (8, _OUT_F))
    return pl.pallas_call(
        _body,
        grid=(b // bt,),
        in_specs=[
            pl.BlockSpec((bt, _IN_F), lambda i: (i, 0)),
            pl.BlockSpec((_P - 1, _IN_F, _OUT_F), lambda i: (0, 0, 0)),
            pl.BlockSpec((8, _OUT_F), lambda i: (0, 0)),
        ],
        out_specs=pl.BlockSpec((bt, _OUT_F), lambda i: (i, 0)),
        out_shape=jax.ShapeDtypeStruct((b, _OUT_F), jnp.float32),
    )(x, v, bias)


# speedup vs baseline: 389.2240x; 1.0052x over previous
import jax
import jax.numpy as jnp
from jax.experimental import pallas as pl

_IN_F = 256
_OUT_F = 64
_P = 32
_WIDTH = 4.0


def _body(x_ref, v_ref, b_ref, o_ref):
    p_max = _P - 1
    xs = (x_ref[...] + _WIDTH / 2.0) * (p_max / _WIDTH)

    dot = lambda a, b: jax.lax.dot(a, b, preferred_element_type=jnp.float32)
    acc = b_ref[0:1, :] + dot(xs, v_ref[0])
    for k in range(1, p_max):
        acc += dot(jnp.maximum(xs - float(k), 0.0), v_ref[k])
    o_ref[...] = acc


def kernel(x, kan_weight):
    b = x.shape[0]
    bt = 2048
    wt = jnp.transpose(kan_weight, (1, 0, 2))
    v = jnp.concatenate(
        [
            (wt[1] - wt[0])[None],
            wt[2:] - 2.0 * wt[1:-1] + wt[:-2],
        ],
        axis=0,
    )
    bias = jnp.broadcast_to(jnp.sum(wt[0], axis=0)[None, :], (8, _OUT_F))
    return pl.pallas_call(
        _body,
        grid=(b // bt,),
        in_specs=[
            pl.BlockSpec((bt, _IN_F), lambda i: (i, 0)),
            pl.BlockSpec((_P - 1, _IN_F, _OUT_F), lambda i: (0, 0, 0)),
            pl.BlockSpec((8, _OUT_F), lambda i: (0, 0)),
        ],
        out_specs=pl.BlockSpec((bt, _OUT_F), lambda i: (i, 0)),
        out_shape=jax.ShapeDtypeStruct((b, _OUT_F), jnp.float32),
    )(x, v, bias)
